# baseline (speedup 1.0000x reference)
import functools

import jax
import jax.numpy as jnp
from jax import lax
from jax.experimental import pallas as pl
from jax.experimental.pallas import tpu as pltpu
from jax.experimental.pallas import tpu_sc as plsc

_NC = 2
_NS = 16
_NW = _NC * _NS
_CH = 128


@functools.lru_cache(maxsize=None)
def _sc_gather_fn(n_chunks_per_worker, d):
    nch = n_chunks_per_worker
    mesh = plsc.VectorSubcoreMesh(core_axis_name="c", subcore_axis_name="s")
    n_blocks = _NW * nch
    out_sds = jax.ShapeDtypeStruct((n_blocks, _CH, d), jnp.float32)

    @functools.partial(
        pl.kernel,
        mesh=mesh,
        out_type=(out_sds, out_sds),
        compiler_params=pltpu.CompilerParams(use_tc_tiling_on_sc=False),
        scratch_types=[
            pltpu.VMEM((nch, _CH), jnp.int32),
            pltpu.VMEM((nch, _CH, d), jnp.float32),
            pltpu.VMEM((nch, _CH, d), jnp.float32),
            pltpu.SemaphoreType.DMA,
        ],
    )
    def sc_gather(x_hbm, ic_hbm, idx_hbm, xg_hbm, icg_hbm,
                  idx_v, xbuf, icbuf, sem):
        wid = lax.axis_index("s") * _NC + lax.axis_index("c")
        base = wid * nch
        pltpu.sync_copy(idx_hbm.at[pl.ds(base, nch)], idx_v)
        copies = []
        for j in range(nch):
            copies.append(
                pltpu.async_copy(x_hbm.at[idx_v.at[j]], xbuf.at[j], sem))
            copies.append(
                pltpu.async_copy(ic_hbm.at[idx_v.at[j]], icbuf.at[j], sem))
        for c in copies:
            c.wait()
        pltpu.sync_copy(xbuf, xg_hbm.at[pl.ds(base, nch)])
        pltpu.sync_copy(icbuf, icg_hbm.at[pl.ds(base, nch)])

    return sc_gather


def _tc_mul_matmul(xg, icg, W, block_rows):
    b, d = xg.shape

    def body(xg_ref, icg_ref, w_ref, o_ref):
        o_ref[...] = jnp.dot(xg_ref[...] * icg_ref[...], w_ref[...],
                             preferred_element_type=jnp.float32)

    return pl.pallas_call(
        body,
        grid=(b // block_rows,),
        in_specs=[
            pl.BlockSpec((block_rows, d), lambda i: (i, 0)),
            pl.BlockSpec((block_rows, d), lambda i: (i, 0)),
            pl.BlockSpec((d, d), lambda i: (0, 0)),
        ],
        out_specs=pl.BlockSpec((block_rows, d), lambda i: (i, 0)),
        out_shape=jax.ShapeDtypeStruct((b, d), jnp.float32),
    )(xg, icg, W)


def kernel(x, input_constant, W, output_constant):
    n, d = x.shape
    b = output_constant.shape[0]
    assert b % (_NW * _CH) == 0
    nch = b // (_NW * _CH)

    idx = output_constant.astype(jnp.int32).reshape(b // _CH, _CH)
    xg3, icg3 = _sc_gather_fn(nch, d)(x, input_constant, idx)
    xg = xg3.reshape(b, d)
    icg = icg3.reshape(b, d)
    return _tc_mul_matmul(xg, icg, W, block_rows=2048)


# speedup vs baseline: 2.3225x; 2.3225x over previous
import functools

import jax
import jax.numpy as jnp
from jax import lax
from jax.experimental import pallas as pl
from jax.experimental.pallas import tpu as pltpu
from jax.experimental.pallas import tpu_sc as plsc

_NC = 2
_NS = 16
_NW = _NC * _NS
_CH = 128
_LP = 128


def _dense_rows(xt, ict, w_pad, block_cols):
    d, n = xt.shape
    grid = (n + block_cols - 1) // block_cols

    def body(xt_ref, ict_ref, w_ref, o_ref):
        ct = xt_ref[...] * ict_ref[...]
        o_ref[...] = lax.dot_general(
            ct, w_ref[...], (((0,), (0,)), ((), ())),
            preferred_element_type=jnp.float32)

    return pl.pallas_call(
        body,
        grid=(grid,),
        in_specs=[
            pl.BlockSpec((d, block_cols), lambda i: (0, i)),
            pl.BlockSpec((d, block_cols), lambda i: (0, i)),
            pl.BlockSpec((d, _LP), lambda i: (0, 0)),
        ],
        out_specs=pl.BlockSpec((block_cols, _LP), lambda i: (i, 0)),
        out_shape=jax.ShapeDtypeStruct((n, _LP), jnp.float32),
        compiler_params=pltpu.CompilerParams(
            dimension_semantics=("parallel",)),
    )(xt, ict, w_pad)


@functools.lru_cache(maxsize=None)
def _sc_gather_fn(n_chunks_per_worker):
    nch = n_chunks_per_worker
    bpw = nch * _CH
    mesh = plsc.VectorSubcoreMesh(core_axis_name="c", subcore_axis_name="s")

    @functools.partial(
        pl.kernel,
        mesh=mesh,
        out_type=jax.ShapeDtypeStruct((_NW * bpw, _LP), jnp.float32),
        scratch_types=[
            pltpu.VMEM((nch, _CH), jnp.int32),
            pltpu.VMEM((bpw, _LP), jnp.float32),
            pltpu.SemaphoreType.DMA,
        ],
    )
    def sc_gather(table_hbm, idx_hbm, out_hbm, idx_v, rows_v, sem):
        wid = lax.axis_index("s") * _NC + lax.axis_index("c")
        pltpu.sync_copy(idx_hbm.at[pl.ds(wid * nch, nch)], idx_v)
        copies = []
        for j in range(nch):
            copies.append(pltpu.async_copy(
                table_hbm.at[idx_v.at[j]],
                rows_v.at[pl.ds(j * _CH, _CH)], sem))
        for c in copies:
            c.wait()
        pltpu.sync_copy(rows_v, out_hbm.at[pl.ds(wid * bpw, bpw)])

    return sc_gather


def kernel(x, input_constant, W, output_constant):
    n, d = x.shape
    b = output_constant.shape[0]
    assert b % (_NW * _CH) == 0
    nch = b // (_NW * _CH)

    xt = x.T
    ict = input_constant.T
    w_pad = jnp.pad(W, ((0, 0), (0, _LP - d)))

    r_pad = _dense_rows(xt, ict, w_pad, block_cols=4096)
    idx = output_constant.astype(jnp.int32).reshape(b // _CH, _CH)
    g = _sc_gather_fn(nch)(r_pad, idx)
    return g[:, :d]


# speedup vs baseline: 2.6213x; 1.1287x over previous
import functools

import jax
import jax.numpy as jnp
from jax import lax
from jax.experimental import pallas as pl
from jax.experimental.pallas import tpu as pltpu
from jax.experimental.pallas import tpu_sc as plsc

_NC = 2
_NS = 16
_NW = _NC * _NS
_CH = 128
_LP = 128


def _dense_rows(xt, ict, w_pad, block_cols):
    d, n = xt.shape
    grid = (n + block_cols - 1) // block_cols

    def body(xt_ref, ict_ref, w_ref, o_ref):
        ct = xt_ref[...] * ict_ref[...]
        o_ref[:, :d] = lax.dot_general(
            ct, w_ref[...], (((0,), (0,)), ((), ())),
            preferred_element_type=jnp.float32)

    return pl.pallas_call(
        body,
        grid=(grid,),
        in_specs=[
            pl.BlockSpec((d, block_cols), lambda i: (0, i)),
            pl.BlockSpec((d, block_cols), lambda i: (0, i)),
            pl.BlockSpec((d, d), lambda i: (0, 0)),
        ],
        out_specs=pl.BlockSpec((block_cols, _LP), lambda i: (i, 0)),
        out_shape=jax.ShapeDtypeStruct((n, _LP), jnp.float32),
        compiler_params=pltpu.CompilerParams(
            dimension_semantics=("parallel",)),
    )(xt, ict, w_pad)


@functools.lru_cache(maxsize=None)
def _sc_gather_fn(n_chunks_per_worker):
    nch = n_chunks_per_worker
    bpw = nch * _CH
    mesh = plsc.VectorSubcoreMesh(core_axis_name="c", subcore_axis_name="s")

    @functools.partial(
        pl.kernel,
        mesh=mesh,
        out_type=jax.ShapeDtypeStruct((_NW * bpw, _LP), jnp.float32),
        scratch_types=[
            pltpu.VMEM((nch, _CH), jnp.int32),
            pltpu.VMEM((bpw, _LP), jnp.float32),
            pltpu.SemaphoreType.DMA,
        ],
    )
    def sc_gather(table_hbm, idx_hbm, out_hbm, idx_v, rows_v, sem):
        wid = lax.axis_index("s") * _NC + lax.axis_index("c")
        pltpu.sync_copy(idx_hbm.at[pl.ds(wid * nch, nch)], idx_v)
        copies = []
        for j in range(nch):
            copies.append(pltpu.async_copy(
                table_hbm.at[idx_v.at[j]],
                rows_v.at[pl.ds(j * _CH, _CH)], sem))
        for c in copies:
            c.wait()
        pltpu.sync_copy(rows_v, out_hbm.at[pl.ds(wid * bpw, bpw)])

    return sc_gather


def kernel(x, input_constant, W, output_constant):
    n, d = x.shape
    b = output_constant.shape[0]
    assert b % (_NW * _CH) == 0
    nch = b // (_NW * _CH)

    xt = x.T
    ict = input_constant.T

    r_pad = _dense_rows(xt, ict, W, block_cols=8192)
    idx = output_constant.astype(jnp.int32).reshape(b // _CH, _CH)
    g = _sc_gather_fn(nch)(r_pad, idx)
    return g[:, :d]
